# Initial kernel scaffold; baseline (speedup 1.0000x reference)
import jax
import jax.numpy as jnp
from jax.experimental import pallas as pl


def kernel(feat, edge_index, feat_src, edge_index_src, feat_tar, edge_index_tar, embed, gcn_w, gcn_b, fc_w, fc_b, def_w, def_b):
    raise NotImplementedError("write your pallas kernel here")


# speedup vs baseline: 5.1304x; 5.1304x over previous
import jax
import jax.numpy as jnp
from jax import lax
from jax.experimental import pallas as pl
from jax.experimental.pallas import tpu as pltpu
from jax.experimental.pallas import tpu_sc as plsc

N_MAIN = 100000
N_SIDE = 50000
E_MAIN = 1600000
E_SIDE = 800000
EMBED_DIM = 32
HIDDEN = 100
HID_PAD = 128

_MESH = plsc.VectorSubcoreMesh(
    core_axis_name="c", subcore_axis_name="s", num_cores=2, num_subcores=16
)

_F32 = jnp.float32
_I32 = jnp.int32


def _fill(ref, n, val, dtype):
    vec = jnp.full((16,), val, dtype)
    for j in range(n // 16):
        ref[pl.ds(j * 16, 16)] = vec


def _deg_body(srcm, dstm, srcs, dsts, srct, dstt,
              odm, idm, ods, ids_, odt, idt,
              spm, idx_v, ones_v, zb, db):
    cid = lax.axis_index("c")
    sid = lax.axis_index("s")

    _fill(ones_v, 128, 1.0, _F32)
    _fill(zb, 2000, 0.0, _F32)

    for k in range(7):
        c = sid + 16 * k

        @pl.when(c < 100)
        def _():
            pltpu.sync_copy(zb, spm.at[pl.ds(pl.multiple_of(c * 2000, 2000), 2000)])

    plsc.subcore_barrier()

    def scatter_deg(edge_ref, off, nchunks):
        trips = (nchunks + 15) // 16

        def body(i, carry):
            c = sid + 16 * i

            @pl.when(c < nchunks)
            def _():
                base = pl.multiple_of(c * 128, 128)
                pltpu.sync_copy(edge_ref.at[pl.ds(base, 128)], idx_v)
                if off:
                    offv = jnp.full((16,), off, _I32)
                    for j in range(8):
                        idx_v[pl.ds(j * 16, 16)] = idx_v[pl.ds(j * 16, 16)] + offv
                pltpu.sync_copy(ones_v, spm.at[idx_v], add=True)

            return carry

        lax.fori_loop(0, trips, body, 0)

    @pl.when(cid == 0)
    def _():
        scatter_deg(srcm, 0, E_MAIN // 128)
        scatter_deg(dstm, 100000, E_MAIN // 128)

    @pl.when(cid == 1)
    def _():
        scatter_deg(srcs, 0, E_SIDE // 128)
        scatter_deg(dsts, 50000, E_SIDE // 128)
        scatter_deg(srct, 100000, E_SIDE // 128)
        scatter_deg(dstt, 150000, E_SIDE // 128)

    plsc.subcore_barrier()

    def drain(off, hbm_ref, n):
        nch = n // 2000
        for k in range((nch + 15) // 16):
            c = sid + 16 * k

            @pl.when(c < nch)
            def _():
                b = pl.multiple_of(c * 2000, 2000)
                pltpu.sync_copy(spm.at[pl.ds(off + b, 2000)], db)
                pltpu.sync_copy(db, hbm_ref.at[pl.ds(b, 2000)])

    @pl.when(cid == 0)
    def _():
        drain(0, odm, N_MAIN)
        drain(100000, idm, N_MAIN)

    @pl.when(cid == 1)
    def _():
        drain(0, ods, N_SIDE)
        drain(50000, ids_, N_SIDE)
        drain(100000, odt, N_SIDE)
        drain(150000, idt, N_SIDE)


_deg_kernel = pl.kernel(
    _deg_body,
    out_type=(
        jax.ShapeDtypeStruct((N_MAIN,), _F32),
        jax.ShapeDtypeStruct((N_MAIN,), _F32),
        jax.ShapeDtypeStruct((N_SIDE,), _F32),
        jax.ShapeDtypeStruct((N_SIDE,), _F32),
        jax.ShapeDtypeStruct((N_SIDE,), _F32),
        jax.ShapeDtypeStruct((N_SIDE,), _F32),
    ),
    mesh=_MESH,
    compiler_params=pltpu.CompilerParams(use_tc_tiling_on_sc=False),
    scratch_types=[
        pltpu.VMEM_SHARED((200000,), _F32),
        pltpu.VMEM((128,), _I32),
        pltpu.VMEM((128,), _F32),
        pltpu.VMEM((2000,), _F32),
        pltpu.VMEM((2000,), _F32),
    ],
)


def _gather_body(featm, feats, featt, embed, xm, xs, xt, idx_v, rows_v, sem):
    cid = lax.axis_index("c")
    sid = lax.axis_index("s")
    gw = sid * 2 + cid

    def phase(feat_ref, x_ref, n):
        nchunks = n // 80
        trips = (nchunks + 31) // 32

        def body(i, carry):
            c = gw + 32 * i

            @pl.when(c < nchunks)
            def _():
                base = pl.multiple_of(c * 80, 80)
                pltpu.sync_copy(feat_ref.at[pl.ds(base, 80)], idx_v)
                pltpu.async_copy(embed.at[idx_v], rows_v, sem).wait()
                pltpu.sync_copy(rows_v, x_ref.at[pl.ds(base, 80)])

            return carry

        lax.fori_loop(0, trips, body, 0)

    phase(featm, xm, N_MAIN)
    phase(feats, xs, N_SIDE)
    phase(featt, xt, N_SIDE)


_gather_kernel = pl.kernel(
    _gather_body,
    out_type=(
        jax.ShapeDtypeStruct((N_MAIN, EMBED_DIM), _F32),
        jax.ShapeDtypeStruct((N_SIDE, EMBED_DIM), _F32),
        jax.ShapeDtypeStruct((N_SIDE, EMBED_DIM), _F32),
    ),
    mesh=_MESH,
    compiler_params=pltpu.CompilerParams(use_tc_tiling_on_sc=False),
    scratch_types=[
        pltpu.VMEM((80,), _I32),
        pltpu.VMEM((80, EMBED_DIM), _F32),
        pltpu.SemaphoreType.DMA,
    ],
)


def _agg_body(hm, hs, ht, srcm, dstm, srcs, dsts, srct, dstt,
              aggm, aggs, aggt,
              spm, sidx, didx, rows, zb, db, sem):
    cid = lax.axis_index("c")
    sid = lax.axis_index("s")

    for j in range(125):
        for k in range(2):
            zb[j, pl.ds(k * 16, 16)] = jnp.zeros((16,), _F32)

    def zero_spm():
        for k in range(25):
            c = sid + 16 * k
            pltpu.sync_copy(zb, spm.at[pl.ds(pl.multiple_of(c * 125, 125), 125)])

    zero_spm()
    plsc.subcore_barrier()

    nchunks = E_MAIN // 128
    trips = (nchunks + 15) // 16
    base_node = jnp.full((16,), cid * N_SIDE, _I32)
    dummy = jnp.full((16,), N_SIDE, _I32)

    def bodyA(i, carry):
        c = sid + 16 * i

        @pl.when(c < nchunks)
        def _():
            base = pl.multiple_of(c * 128, 128)
            pltpu.sync_copy(srcm.at[pl.ds(base, 128)], sidx)
            pltpu.sync_copy(dstm.at[pl.ds(base, 128)], didx)
            pltpu.async_copy(hm.at[sidx], rows, sem).wait()
            for j in range(8):
                d = didx[pl.ds(j * 16, 16)]
                inr = (d >= base_node) & (d < base_node + N_SIDE)
                didx[pl.ds(j * 16, 16)] = jnp.where(inr, d - base_node, dummy)
            pltpu.sync_copy(rows, spm.at[didx], add=True)

        return carry

    lax.fori_loop(0, trips, bodyA, 0)
    plsc.subcore_barrier()

    for k in range(25):
        r0 = sid * 3125 + k * 125
        pltpu.sync_copy(spm.at[pl.ds(r0, 125)], db)
        pltpu.sync_copy(db, aggm.at[pl.ds(cid * N_SIDE + r0, 125)])

    plsc.subcore_barrier()
    zero_spm()
    plsc.subcore_barrier()

    nchunksB = E_SIDE // 128
    tripsB = (nchunksB + 15) // 16

    def side(src_ref, dst_ref, h_ref):
        def bodyB(i, carry):
            c = sid + 16 * i

            @pl.when(c < nchunksB)
            def _():
                base = pl.multiple_of(c * 128, 128)
                pltpu.sync_copy(src_ref.at[pl.ds(base, 128)], sidx)
                pltpu.sync_copy(dst_ref.at[pl.ds(base, 128)], didx)
                pltpu.async_copy(h_ref.at[sidx], rows, sem).wait()
                pltpu.sync_copy(rows, spm.at[didx], add=True)

            return carry

        lax.fori_loop(0, tripsB, bodyB, 0)

    @pl.when(cid == 0)
    def _():
        side(srcs, dsts, hs)

    @pl.when(cid == 1)
    def _():
        side(srct, dstt, ht)

    plsc.subcore_barrier()

    def drain_side(agg_ref):
        for k in range(25):
            r0 = sid * 3125 + k * 125
            pltpu.sync_copy(spm.at[pl.ds(r0, 125)], db)
            pltpu.sync_copy(db, agg_ref.at[pl.ds(r0, 125)])

    @pl.when(cid == 0)
    def _():
        drain_side(aggs)

    @pl.when(cid == 1)
    def _():
        drain_side(aggt)


_agg_kernel = pl.kernel(
    _agg_body,
    out_type=(
        jax.ShapeDtypeStruct((N_MAIN, EMBED_DIM), _F32),
        jax.ShapeDtypeStruct((N_SIDE, EMBED_DIM), _F32),
        jax.ShapeDtypeStruct((N_SIDE, EMBED_DIM), _F32),
    ),
    mesh=_MESH,
    compiler_params=pltpu.CompilerParams(use_tc_tiling_on_sc=False),
    scratch_types=[
        pltpu.VMEM_SHARED((N_SIDE + 1, EMBED_DIM), _F32),
        pltpu.VMEM((128,), _I32),
        pltpu.VMEM((128,), _I32),
        pltpu.VMEM((128, EMBED_DIM), _F32),
        pltpu.VMEM((125, EMBED_DIM), _F32),
        pltpu.VMEM((125, EMBED_DIM), _F32),
        pltpu.SemaphoreType.DMA,
    ],
)


def _scale_body(x_ref, d_ref, o_ref):
    d = d_ref[...]
    norm = jnp.where(d > 0, lax.rsqrt(jnp.maximum(d, 1.0)), 0.0)
    o_ref[...] = x_ref[...] * norm


def _scale(x, deg, n):
    blk = 1000
    return pl.pallas_call(
        _scale_body,
        grid=(n // blk,),
        in_specs=[
            pl.BlockSpec((blk, EMBED_DIM), lambda i: (i, 0)),
            pl.BlockSpec((blk, 1), lambda i: (i, 0)),
        ],
        out_specs=pl.BlockSpec((blk, EMBED_DIM), lambda i: (i, 0)),
        out_shape=jax.ShapeDtypeStruct((n, EMBED_DIM), _F32),
    )(x, deg.reshape(n, 1))


def _mm_body(agg_ref, d_ref, w_ref, b_ref, o_ref, acc_ref):
    i = pl.program_id(0)
    d = d_ref[...]
    norm = jnp.where(d > 0, lax.rsqrt(jnp.maximum(d, 1.0)), 0.0)
    z = jnp.dot(agg_ref[...] * norm, w_ref[...], preferred_element_type=_F32)
    z = jnp.maximum(z + b_ref[...], 0.0)
    m = jnp.max(z, axis=0, keepdims=True)

    @pl.when(i == 0)
    def _():
        acc_ref[...] = m

    @pl.when(i > 0)
    def _():
        acc_ref[...] = jnp.maximum(acc_ref[...], m)

    @pl.when(i == pl.num_programs(0) - 1)
    def _():
        o_ref[...] = acc_ref[...]


def _max_pool(agg, indeg, n, w_p, b_p):
    blk = 1000
    return pl.pallas_call(
        _mm_body,
        grid=(n // blk,),
        in_specs=[
            pl.BlockSpec((blk, EMBED_DIM), lambda i: (i, 0)),
            pl.BlockSpec((blk, 1), lambda i: (i, 0)),
            pl.BlockSpec((EMBED_DIM, HID_PAD), lambda i: (0, 0)),
            pl.BlockSpec((1, HID_PAD), lambda i: (0, 0)),
        ],
        out_specs=pl.BlockSpec((1, HID_PAD), lambda i: (0, 0)),
        out_shape=jax.ShapeDtypeStruct((1, HID_PAD), _F32),
        scratch_shapes=[pltpu.VMEM((1, HID_PAD), _F32)],
    )(agg, indeg.reshape(n, 1), w_p, b_p)


def _head_body(xm_ref, gs_ref, gt_ref, fcw_ref, fcb_ref, dw_ref, db_ref,
               o_b, o_d, o_s, o_t):
    fcw = fcw_ref[...]
    fcb = fcb_ref[...]
    xb = jnp.dot(xm_ref[...], fcw, preferred_element_type=_F32) + fcb
    xs = jnp.dot(gs_ref[...], fcw, preferred_element_type=_F32) + fcb
    xt = jnp.dot(gt_ref[...], fcw, preferred_element_type=_F32) + fcb
    logit = jnp.dot(xb, dw_ref[...], preferred_element_type=_F32) + db_ref[...]
    o_b[...] = xb
    o_d[...] = 1.0 / (1.0 + jnp.exp(-logit))
    o_s[...] = xs
    o_t[...] = xt


def _head(xm, gs, gt, fcw_p, fcb2, dw, db2):
    return pl.pallas_call(
        _head_body,
        out_shape=(
            jax.ShapeDtypeStruct((1, 32), _F32),
            jax.ShapeDtypeStruct((1, 1), _F32),
            jax.ShapeDtypeStruct((1, 32), _F32),
            jax.ShapeDtypeStruct((1, 32), _F32),
        ),
    )(xm, gs, gt, fcw_p, fcb2, dw, db2)


def kernel(feat, edge_index, feat_src, edge_index_src, feat_tar, edge_index_tar,
           embed, gcn_w, gcn_b, fc_w, fc_b, def_w, def_b):
    srcm, dstm = edge_index[0], edge_index[1]
    srcs, dsts = edge_index_src[0], edge_index_src[1]
    srct, dstt = edge_index_tar[0], edge_index_tar[1]

    odm, idm, ods, ids_, odt, idt = _deg_kernel(srcm, dstm, srcs, dsts, srct, dstt)
    xm, xs, xt = _gather_kernel(feat, feat_src, feat_tar, embed)

    hm = _scale(xm, odm, N_MAIN)
    hs = _scale(xs, ods, N_SIDE)
    ht = _scale(xt, odt, N_SIDE)

    aggm, aggs, aggt = _agg_kernel(hm, hs, ht, srcm, dstm, srcs, dsts, srct, dstt)

    w_p = jnp.pad(gcn_w, ((0, 0), (0, HID_PAD - HIDDEN)))
    b_p = jnp.pad(gcn_b, (0, HID_PAD - HIDDEN)).reshape(1, HID_PAD)

    x_max = _max_pool(aggm, idm, N_MAIN, w_p, b_p)
    g_src = _max_pool(aggs, ids_, N_SIDE, w_p, b_p)
    g_tar = _max_pool(aggt, idt, N_SIDE, w_p, b_p)

    fcw_p = jnp.pad(fc_w, ((0, HID_PAD - HIDDEN), (0, 0)))
    fcb2 = fc_b.reshape(1, 32)
    db2 = def_b.reshape(1, 1)

    x_batch_mmd, defect_prob, x_src_mmd, x_tar_mmd = _head(
        x_max, g_src, g_tar, fcw_p, fcb2, def_w, db2
    )
    return (x_batch_mmd, defect_prob, x_src_mmd, x_tar_mmd)


# speedup vs baseline: 9.6824x; 1.8873x over previous
import jax
import jax.numpy as jnp
from jax import lax
from jax.experimental import pallas as pl
from jax.experimental.pallas import tpu as pltpu
from jax.experimental.pallas import tpu_sc as plsc

N_MAIN = 100000
N_SIDE = 50000
N_ALL = N_MAIN + 2 * N_SIDE
E_MAIN = 1600000
E_SIDE = 800000
EMBED_DIM = 32
HIDDEN = 100
HID_PAD = 128

J = 5
SUP = J * 128
J3 = 4
SUB3 = 80
SUP3 = J3 * SUB3
GB = 80

_MESH = plsc.VectorSubcoreMesh(
    core_axis_name="c", subcore_axis_name="s", num_cores=2, num_subcores=16
)

_F32 = jnp.float32
_I32 = jnp.int32


def _fill(ref, n, val, dtype):
    vec = jnp.full((16,), val, dtype)
    for j in range(n // 16):
        ref[pl.ds(j * 16, 16)] = vec


def _deg_body(srcm, dstm, srcs, dsts, srct, dstt,
              od_all, id_all,
              spm, il, isx, ones_v, zb, db, si0, si1, ss0, ss1):
    cid = lax.axis_index("c")
    sid = lax.axis_index("s")
    sem_i = [si0, si1]
    sem_s = [ss0, ss1]

    _fill(ones_v, 128, 1.0, _F32)
    _fill(zb, 2000, 0.0, _F32)

    for k in range(7):
        c = sid + 16 * k

        @pl.when(c < 100)
        def _():
            pltpu.sync_copy(zb, spm.at[pl.ds(pl.multiple_of(c * 2000, 2000), 2000)])

    plsc.subcore_barrier()

    def scatter_deg(edge_ref, off, nsuper):
        offv = jnp.full((16,), off, _I32)

        def issue_idx(c, s):
            @pl.when(c < nsuper)
            def _():
                b = pl.multiple_of(c * SUP, 128)
                for j in range(J):
                    pltpu.async_copy(
                        edge_ref.at[pl.ds(b + j * 128, 128)], il.at[s, j], sem_i[s]
                    )

        def wait_idx(c, s):
            @pl.when(c < nsuper)
            def _():
                b = pl.multiple_of(c * SUP, 128)
                for j in range(J):
                    pltpu.make_async_copy(
                        edge_ref.at[pl.ds(b + j * 128, 128)], il.at[s, j], sem_i[s]
                    ).wait()

        def process(c, s):
            @pl.when(c < nsuper)
            def _():
                for j in range(J):
                    for q in range(8):
                        isx[s, j, pl.ds(q * 16, 16)] = (
                            il[s, j, pl.ds(q * 16, 16)] + offv
                        )
                for j in range(J):
                    pltpu.async_copy(
                        ones_v, spm.at[isx.at[s, j]], sem_s[s], add=True
                    )

        def wait_sc(c, s):
            @pl.when(c < nsuper)
            def _():
                for j in range(J):
                    pltpu.make_async_copy(
                        ones_v, spm.at[isx.at[s, j]], sem_s[s]
                    ).wait()

        trips = (nsuper + 15) // 16
        trips2 = (trips + 1) // 2

        issue_idx(sid, 0)

        def lbody(k2, carry):
            c0 = sid + 16 * (2 * k2)
            c1 = c0 + 16
            c2 = c0 + 32
            wait_idx(c0, 0)
            issue_idx(c1, 1)

            @pl.when(k2 > 0)
            def _():
                wait_sc(c0 - 32, 0)

            process(c0, 0)
            wait_idx(c1, 1)
            issue_idx(c2, 0)

            @pl.when(k2 > 0)
            def _():
                wait_sc(c1 - 32, 1)

            process(c1, 1)
            return carry

        lax.fori_loop(0, trips2, lbody, 0)
        wait_sc(sid + 16 * (2 * trips2 - 2), 0)
        wait_sc(sid + 16 * (2 * trips2 - 1), 1)

    @pl.when(cid == 0)
    def _():
        scatter_deg(srcm, 0, E_MAIN // SUP)
        scatter_deg(dstm, 100000, E_MAIN // SUP)

    @pl.when(cid == 1)
    def _():
        scatter_deg(srcs, 0, E_SIDE // SUP)
        scatter_deg(dsts, 50000, E_SIDE // SUP)
        scatter_deg(srct, 100000, E_SIDE // SUP)
        scatter_deg(dstt, 150000, E_SIDE // SUP)

    plsc.subcore_barrier()

    def drain(spm_off, hbm_ref, hbm_off, n):
        nch = n // 2000
        for k in range((nch + 15) // 16):
            c = sid + 16 * k

            @pl.when(c < nch)
            def _():
                b = pl.multiple_of(c * 2000, 2000)
                pltpu.sync_copy(spm.at[pl.ds(spm_off + b, 2000)], db)
                pltpu.sync_copy(db, hbm_ref.at[pl.ds(hbm_off + b, 2000)])

    @pl.when(cid == 0)
    def _():
        drain(0, od_all, 0, N_MAIN)
        drain(100000, id_all, 0, N_MAIN)

    @pl.when(cid == 1)
    def _():
        drain(0, od_all, 100000, N_SIDE)
        drain(50000, id_all, 100000, N_SIDE)
        drain(100000, od_all, 150000, N_SIDE)
        drain(150000, id_all, 150000, N_SIDE)


_deg_kernel = pl.kernel(
    _deg_body,
    out_type=(
        jax.ShapeDtypeStruct((N_ALL,), _F32),
        jax.ShapeDtypeStruct((N_ALL,), _F32),
    ),
    mesh=_MESH,
    compiler_params=pltpu.CompilerParams(use_tc_tiling_on_sc=False),
    scratch_types=[
        pltpu.VMEM_SHARED((200000,), _F32),
        pltpu.VMEM((2, J, 128), _I32),
        pltpu.VMEM((2, J, 128), _I32),
        pltpu.VMEM((128,), _F32),
        pltpu.VMEM((2000,), _F32),
        pltpu.VMEM((2000,), _F32),
        pltpu.SemaphoreType.DMA,
        pltpu.SemaphoreType.DMA,
        pltpu.SemaphoreType.DMA,
        pltpu.SemaphoreType.DMA,
    ],
)


def _h_body(featm, feats, featt, embed, x_all, idx_v, rows_v, sem):
    cid = lax.axis_index("c")
    sid = lax.axis_index("s")
    gw = sid * 2 + cid

    def phase(feat_ref, n, x_off):
        nchunks = n // GB
        trips = (nchunks + 31) // 32

        def body(i, carry):
            c = gw + 32 * i

            @pl.when(c < nchunks)
            def _():
                base = pl.multiple_of(c * GB, GB)
                pltpu.sync_copy(feat_ref.at[pl.ds(base, GB)], idx_v)
                pltpu.async_copy(embed.at[idx_v], rows_v, sem).wait()
                pltpu.sync_copy(rows_v, x_all.at[pl.ds(x_off + base, GB)])

            return carry

        lax.fori_loop(0, trips, body, 0)

    phase(featm, N_MAIN, 0)
    phase(feats, N_SIDE, 100000)
    phase(featt, N_SIDE, 150000)


_h_kernel = pl.kernel(
    _h_body,
    out_type=jax.ShapeDtypeStruct((N_ALL, EMBED_DIM), _F32),
    mesh=_MESH,
    compiler_params=pltpu.CompilerParams(use_tc_tiling_on_sc=False),
    scratch_types=[
        pltpu.VMEM((GB,), _I32),
        pltpu.VMEM((GB, EMBED_DIM), _F32),
        pltpu.SemaphoreType.DMA,
    ],
)


def _scale_body(x_ref, d_ref, o_ref):
    d = d_ref[...]
    norm = jnp.where(d > 0, lax.rsqrt(jnp.maximum(d, 1.0)), 0.0)
    o_ref[...] = x_ref[...] * norm


def _scale(x_all, od_all):
    return pl.pallas_call(
        _scale_body,
        grid=(N_ALL // _BLK,),
        in_specs=[
            pl.BlockSpec((_BLK, EMBED_DIM), lambda i: (i, 0)),
            pl.BlockSpec((_BLK, 1), lambda i: (i, 0)),
        ],
        out_specs=pl.BlockSpec((_BLK, EMBED_DIM), lambda i: (i, 0)),
        out_shape=jax.ShapeDtypeStruct((N_ALL, EMBED_DIM), _F32),
    )(x_all, od_all.reshape(N_ALL, 1))


def _agg_body(h_all, srcm, dstm, srcs, dsts, srct, dstt,
              agg_all,
              spm, sidx, didxl, didxs, rows, db,
              si0, si1, sg0, sg1, ss0, ss1):
    cid = lax.axis_index("c")
    sid = lax.axis_index("s")
    sem_i = [si0, si1]
    sem_g = [sg0, sg1]
    sem_s = [ss0, ss1]

    def zero_spm():
        for j in range(125):
            for k in range(2):
                db[j, pl.ds(k * 16, 16)] = jnp.zeros((16,), _F32)
        for k in range(25):
            c = sid + 16 * k
            pltpu.sync_copy(db, spm.at[pl.ds(pl.multiple_of(c * 125, 125), 125)])

    zero_spm()
    plsc.subcore_barrier()

    def phase(src_ref, dst_ref, nsuper, masked, src_off):
        base_vec = jnp.full((16,), cid * N_SIDE, _I32)
        dummy = jnp.full((16,), N_SIDE, _I32)
        soffv = jnp.full((16,), src_off, _I32)

        def issue_idx(c, s):
            @pl.when(c < nsuper)
            def _():
                b = pl.multiple_of(c * SUP3, SUP3)
                for j in range(J3):
                    pltpu.async_copy(
                        src_ref.at[pl.ds(b + j * SUB3, SUB3)], sidx.at[s, j], sem_i[s]
                    )
                    pltpu.async_copy(
                        dst_ref.at[pl.ds(b + j * SUB3, SUB3)], didxl.at[s, j], sem_i[s]
                    )

        def wait_idx(c, s):
            @pl.when(c < nsuper)
            def _():
                b = pl.multiple_of(c * SUP3, SUP3)
                for j in range(J3):
                    pltpu.make_async_copy(
                        src_ref.at[pl.ds(b + j * SUB3, SUB3)], sidx.at[s, j], sem_i[s]
                    ).wait()
                    pltpu.make_async_copy(
                        dst_ref.at[pl.ds(b + j * SUB3, SUB3)], didxl.at[s, j], sem_i[s]
                    ).wait()

        def process(c, s):
            @pl.when(c < nsuper)
            def _():
                if src_off:
                    for j in range(J3):
                        for q in range(SUB3 // 16):
                            sidx[s, j, pl.ds(q * 16, 16)] = (
                                sidx[s, j, pl.ds(q * 16, 16)] + soffv
                            )
                for j in range(J3):
                    pltpu.async_copy(h_all.at[sidx.at[s, j]], rows.at[s, j], sem_g[s])
                for j in range(J3):
                    for q in range(SUB3 // 16):
                        d = didxl[s, j, pl.ds(q * 16, 16)]
                        if masked:
                            inr = (d >= base_vec) & (d < base_vec + N_SIDE)
                            d = jnp.where(inr, d - base_vec, dummy)
                        didxs[s, j, pl.ds(q * 16, 16)] = d
                for j in range(J3):
                    pltpu.make_async_copy(
                        h_all.at[sidx.at[s, j]], rows.at[s, j], sem_g[s]
                    ).wait()
                for j in range(J3):
                    pltpu.async_copy(
                        rows.at[s, j], spm.at[didxs.at[s, j]], sem_s[s], add=True
                    )

        def wait_sc(c, s):
            @pl.when(c < nsuper)
            def _():
                for j in range(J3):
                    pltpu.make_async_copy(
                        rows.at[s, j], spm.at[didxs.at[s, j]], sem_s[s]
                    ).wait()

        trips = (nsuper + 15) // 16
        trips2 = (trips + 1) // 2

        issue_idx(sid, 0)

        def lbody(k2, carry):
            c0 = sid + 16 * (2 * k2)
            c1 = c0 + 16
            c2 = c0 + 32
            wait_idx(c0, 0)
            issue_idx(c1, 1)

            @pl.when(k2 > 0)
            def _():
                wait_sc(c0 - 32, 0)

            process(c0, 0)
            wait_idx(c1, 1)
            issue_idx(c2, 0)

            @pl.when(k2 > 0)
            def _():
                wait_sc(c1 - 32, 1)

            process(c1, 1)
            return carry

        lax.fori_loop(0, trips2, lbody, 0)
        wait_sc(sid + 16 * (2 * trips2 - 2), 0)
        wait_sc(sid + 16 * (2 * trips2 - 1), 1)

    phase(srcm, dstm, E_MAIN // SUP3, True, 0)
    plsc.subcore_barrier()

    for k in range(25):
        r0 = sid * 3125 + k * 125
        pltpu.sync_copy(spm.at[pl.ds(r0, 125)], db)
        pltpu.sync_copy(db, agg_all.at[pl.ds(cid * N_SIDE + r0, 125)])

    plsc.subcore_barrier()
    zero_spm()
    plsc.subcore_barrier()

    @pl.when(cid == 0)
    def _():
        phase(srcs, dsts, E_SIDE // SUP3, False, N_MAIN)

    @pl.when(cid == 1)
    def _():
        phase(srct, dstt, E_SIDE // SUP3, False, N_MAIN + N_SIDE)

    plsc.subcore_barrier()

    def drain_side(agg_off):
        for k in range(25):
            r0 = sid * 3125 + k * 125
            pltpu.sync_copy(spm.at[pl.ds(r0, 125)], db)
            pltpu.sync_copy(db, agg_all.at[pl.ds(agg_off + r0, 125)])

    @pl.when(cid == 0)
    def _():
        drain_side(N_MAIN)

    @pl.when(cid == 1)
    def _():
        drain_side(N_MAIN + N_SIDE)


_agg_kernel = pl.kernel(
    _agg_body,
    out_type=jax.ShapeDtypeStruct((N_ALL, EMBED_DIM), _F32),
    mesh=_MESH,
    compiler_params=pltpu.CompilerParams(use_tc_tiling_on_sc=False),
    scratch_types=[
        pltpu.VMEM_SHARED((N_SIDE + 1, EMBED_DIM), _F32),
        pltpu.VMEM((2, J3, SUB3), _I32),
        pltpu.VMEM((2, J3, SUB3), _I32),
        pltpu.VMEM((2, J3, SUB3), _I32),
        pltpu.VMEM((2, J3, SUB3, EMBED_DIM), _F32),
        pltpu.VMEM((125, EMBED_DIM), _F32),
        pltpu.SemaphoreType.DMA,
        pltpu.SemaphoreType.DMA,
        pltpu.SemaphoreType.DMA,
        pltpu.SemaphoreType.DMA,
        pltpu.SemaphoreType.DMA,
        pltpu.SemaphoreType.DMA,
    ],
)


_BLK = 1000
_G_MAIN = N_MAIN // _BLK
_G_SRC = _G_MAIN + N_SIDE // _BLK
_G_ALL = N_ALL // _BLK


def _mm_body(agg_ref, d_ref, w_ref, b_ref, fcw_ref, fcb_ref, dw_ref, db_ref,
             o_b, o_d, o_s, o_t, acc_ref):
    i = pl.program_id(0)
    d = d_ref[...]
    norm = jnp.where(d > 0, lax.rsqrt(jnp.maximum(d, 1.0)), 0.0)
    z = jnp.dot(agg_ref[...] * norm, w_ref[...], preferred_element_type=_F32)
    z = jnp.maximum(z + b_ref[...], 0.0)
    m = jnp.max(z, axis=0, keepdims=True)

    for gid, (lo, hi) in enumerate(((0, _G_MAIN), (_G_MAIN, _G_SRC),
                                    (_G_SRC, _G_ALL))):
        @pl.when(i == lo)
        def _():
            acc_ref[pl.ds(gid, 1), :] = m

        @pl.when((i > lo) & (i < hi))
        def _():
            acc_ref[pl.ds(gid, 1), :] = jnp.maximum(acc_ref[pl.ds(gid, 1), :], m)

    @pl.when(i == _G_ALL - 1)
    def _():
        fcw = fcw_ref[...]
        fcb = fcb_ref[...]
        xb = jnp.dot(acc_ref[pl.ds(0, 1), :], fcw, preferred_element_type=_F32) + fcb
        xs = jnp.dot(acc_ref[pl.ds(1, 1), :], fcw, preferred_element_type=_F32) + fcb
        xt = jnp.dot(acc_ref[pl.ds(2, 1), :], fcw, preferred_element_type=_F32) + fcb
        logit = jnp.dot(xb, dw_ref[...], preferred_element_type=_F32) + db_ref[...]
        o_b[...] = xb
        o_d[...] = 1.0 / (1.0 + jnp.exp(-logit))
        o_s[...] = xs
        o_t[...] = xt


def _max_head(agg_all, id_all, w_p, b_p, fcw_p, fcb2, dw, db2):
    return pl.pallas_call(
        _mm_body,
        grid=(_G_ALL,),
        in_specs=[
            pl.BlockSpec((_BLK, EMBED_DIM), lambda i: (i, 0)),
            pl.BlockSpec((_BLK, 1), lambda i: (i, 0)),
            pl.BlockSpec((EMBED_DIM, HID_PAD), lambda i: (0, 0)),
            pl.BlockSpec((1, HID_PAD), lambda i: (0, 0)),
            pl.BlockSpec((HID_PAD, 32), lambda i: (0, 0)),
            pl.BlockSpec((1, 32), lambda i: (0, 0)),
            pl.BlockSpec((32, 1), lambda i: (0, 0)),
            pl.BlockSpec((1, 1), lambda i: (0, 0)),
        ],
        out_specs=(
            pl.BlockSpec((1, 32), lambda i: (0, 0)),
            pl.BlockSpec((1, 1), lambda i: (0, 0)),
            pl.BlockSpec((1, 32), lambda i: (0, 0)),
            pl.BlockSpec((1, 32), lambda i: (0, 0)),
        ),
        out_shape=(
            jax.ShapeDtypeStruct((1, 32), _F32),
            jax.ShapeDtypeStruct((1, 1), _F32),
            jax.ShapeDtypeStruct((1, 32), _F32),
            jax.ShapeDtypeStruct((1, 32), _F32),
        ),
        scratch_shapes=[pltpu.VMEM((8, HID_PAD), _F32)],
    )(agg_all, id_all.reshape(N_ALL, 1), w_p, b_p, fcw_p, fcb2, dw, db2)


def kernel(feat, edge_index, feat_src, edge_index_src, feat_tar, edge_index_tar,
           embed, gcn_w, gcn_b, fc_w, fc_b, def_w, def_b):
    srcm, dstm = edge_index[0], edge_index[1]
    srcs, dsts = edge_index_src[0], edge_index_src[1]
    srct, dstt = edge_index_tar[0], edge_index_tar[1]

    od_all, id_all = _deg_kernel(srcm, dstm, srcs, dsts, srct, dstt)
    x_all = _h_kernel(feat, feat_src, feat_tar, embed)
    h_all = _scale(x_all, od_all)
    agg_all = _agg_kernel(h_all, srcm, dstm, srcs, dsts, srct, dstt)

    w_p = jnp.pad(gcn_w, ((0, 0), (0, HID_PAD - HIDDEN)))
    b_p = jnp.pad(gcn_b, (0, HID_PAD - HIDDEN)).reshape(1, HID_PAD)
    fcw_p = jnp.pad(fc_w, ((0, HID_PAD - HIDDEN), (0, 0)))
    fcb2 = fc_b.reshape(1, 32)
    db2 = def_b.reshape(1, 1)

    x_batch_mmd, defect_prob, x_src_mmd, x_tar_mmd = _max_head(
        agg_all, id_all, w_p, b_p, fcw_p, fcb2, def_w, db2
    )
    return (x_batch_mmd, defect_prob, x_src_mmd, x_tar_mmd)


# speedup vs baseline: 11.0108x; 1.1372x over previous
import jax
import jax.numpy as jnp
from jax import lax
from jax.experimental import pallas as pl
from jax.experimental.pallas import tpu as pltpu
from jax.experimental.pallas import tpu_sc as plsc

N_MAIN = 100000
N_SIDE = 50000
N_ALL = N_MAIN + 2 * N_SIDE
E_MAIN = 1600000
E_SIDE = 800000
EMBED_DIM = 32
HIDDEN = 100
HID_PAD = 128

J = 5
SUP = J * 128
J3 = 4
SUB3 = 80
SUP3 = J3 * SUB3
GB = 80

_MESH = plsc.VectorSubcoreMesh(
    core_axis_name="c", subcore_axis_name="s", num_cores=2, num_subcores=16
)

_F32 = jnp.float32
_I32 = jnp.int32


def _fill(ref, n, val, dtype):
    vec = jnp.full((16,), val, dtype)
    for j in range(n // 16):
        ref[pl.ds(j * 16, 16)] = vec


def _deg_body(eim, eis, eit,
              od_all, id_all,
              spm, il, isx, ones_v, zb, db, si0, si1, ss0, ss1):
    cid = lax.axis_index("c")
    sid = lax.axis_index("s")
    sem_i = [si0, si1]
    sem_s = [ss0, ss1]

    _fill(ones_v, 128, 1.0, _F32)
    _fill(zb, 2000, 0.0, _F32)

    for k in range(7):
        c = sid + 16 * k

        @pl.when(c < 100)
        def _():
            pltpu.sync_copy(zb, spm.at[pl.ds(pl.multiple_of(c * 2000, 2000), 2000)])

    plsc.subcore_barrier()

    def scatter_deg(edge_ref, row, off, nsuper):
        offv = jnp.full((16,), off, _I32)

        def issue_idx(c, s):
            @pl.when(c < nsuper)
            def _():
                b = pl.multiple_of(c * SUP, 128)
                for j in range(J):
                    pltpu.async_copy(
                        edge_ref.at[row, pl.ds(b + j * 128, 128)], il.at[s, j],
                        sem_i[s],
                    )

        def wait_idx(c, s):
            @pl.when(c < nsuper)
            def _():
                b = pl.multiple_of(c * SUP, 128)
                for j in range(J):
                    pltpu.make_async_copy(
                        edge_ref.at[row, pl.ds(b + j * 128, 128)], il.at[s, j],
                        sem_i[s],
                    ).wait()

        def process(c, s):
            @pl.when(c < nsuper)
            def _():
                for j in range(J):
                    for q in range(8):
                        isx[s, j, pl.ds(q * 16, 16)] = (
                            il[s, j, pl.ds(q * 16, 16)] + offv
                        )
                for j in range(J):
                    pltpu.async_copy(
                        ones_v, spm.at[isx.at[s, j]], sem_s[s], add=True
                    )

        def wait_sc(c, s):
            @pl.when(c < nsuper)
            def _():
                for j in range(J):
                    pltpu.make_async_copy(
                        ones_v, spm.at[isx.at[s, j]], sem_s[s]
                    ).wait()

        trips = (nsuper + 15) // 16
        trips2 = (trips + 1) // 2

        issue_idx(sid, 0)

        def lbody(k2, carry):
            c0 = sid + 16 * (2 * k2)
            c1 = c0 + 16
            c2 = c0 + 32
            wait_idx(c0, 0)
            issue_idx(c1, 1)

            @pl.when(k2 > 0)
            def _():
                wait_sc(c0 - 32, 0)

            process(c0, 0)
            wait_idx(c1, 1)
            issue_idx(c2, 0)

            @pl.when(k2 > 0)
            def _():
                wait_sc(c1 - 32, 1)

            process(c1, 1)
            return carry

        lax.fori_loop(0, trips2, lbody, 0)
        wait_sc(sid + 16 * (2 * trips2 - 2), 0)
        wait_sc(sid + 16 * (2 * trips2 - 1), 1)

    @pl.when(cid == 0)
    def _():
        scatter_deg(eim, 0, 0, E_MAIN // SUP)
        scatter_deg(eim, 1, 100000, E_MAIN // SUP)

    @pl.when(cid == 1)
    def _():
        scatter_deg(eis, 0, 0, E_SIDE // SUP)
        scatter_deg(eis, 1, 50000, E_SIDE // SUP)
        scatter_deg(eit, 0, 100000, E_SIDE // SUP)
        scatter_deg(eit, 1, 150000, E_SIDE // SUP)

    plsc.subcore_barrier()

    def drain(spm_off, hbm_ref, hbm_off, n):
        nch = n // 2000
        for k in range((nch + 15) // 16):
            c = sid + 16 * k

            @pl.when(c < nch)
            def _():
                b = pl.multiple_of(c * 2000, 2000)
                pltpu.sync_copy(spm.at[pl.ds(spm_off + b, 2000)], db)
                pltpu.sync_copy(db, hbm_ref.at[pl.ds(hbm_off + b, 2000)])

    @pl.when(cid == 0)
    def _():
        drain(0, od_all, 0, N_MAIN)
        drain(100000, id_all, 0, N_MAIN)

    @pl.when(cid == 1)
    def _():
        drain(0, od_all, 100000, N_SIDE)
        drain(50000, id_all, 100000, N_SIDE)
        drain(100000, od_all, 150000, N_SIDE)
        drain(150000, id_all, 150000, N_SIDE)


_deg_kernel = pl.kernel(
    _deg_body,
    out_type=(
        jax.ShapeDtypeStruct((N_ALL,), _F32),
        jax.ShapeDtypeStruct((N_ALL,), _F32),
    ),
    mesh=_MESH,
    compiler_params=pltpu.CompilerParams(use_tc_tiling_on_sc=False),
    scratch_types=[
        pltpu.VMEM_SHARED((200000,), _F32),
        pltpu.VMEM((2, J, 128), _I32),
        pltpu.VMEM((2, J, 128), _I32),
        pltpu.VMEM((128,), _F32),
        pltpu.VMEM((2000,), _F32),
        pltpu.VMEM((2000,), _F32),
        pltpu.SemaphoreType.DMA,
        pltpu.SemaphoreType.DMA,
        pltpu.SemaphoreType.DMA,
        pltpu.SemaphoreType.DMA,
    ],
)


def _h_body(featm, feats, featt, embed, x_all, idx_v, rows_v, sem):
    cid = lax.axis_index("c")
    sid = lax.axis_index("s")
    gw = sid * 2 + cid

    def phase(feat_ref, n, x_off):
        nchunks = n // GB
        trips = (nchunks + 31) // 32

        def body(i, carry):
            c = gw + 32 * i

            @pl.when(c < nchunks)
            def _():
                base = pl.multiple_of(c * GB, GB)
                pltpu.sync_copy(feat_ref.at[pl.ds(base, GB)], idx_v)
                pltpu.async_copy(embed.at[idx_v], rows_v, sem).wait()
                pltpu.sync_copy(rows_v, x_all.at[pl.ds(x_off + base, GB)])

            return carry

        lax.fori_loop(0, trips, body, 0)

    phase(featm, N_MAIN, 0)
    phase(feats, N_SIDE, 100000)
    phase(featt, N_SIDE, 150000)


_h_kernel = pl.kernel(
    _h_body,
    out_type=jax.ShapeDtypeStruct((N_ALL, EMBED_DIM), _F32),
    mesh=_MESH,
    compiler_params=pltpu.CompilerParams(use_tc_tiling_on_sc=False),
    scratch_types=[
        pltpu.VMEM((GB,), _I32),
        pltpu.VMEM((GB, EMBED_DIM), _F32),
        pltpu.SemaphoreType.DMA,
    ],
)


def _scale_body(x_ref, d_ref, o_ref):
    d = d_ref[...]
    norm = jnp.where(d > 0, lax.rsqrt(jnp.maximum(d, 1.0)), 0.0)
    o_ref[...] = x_ref[...] * norm


def _scale(x_all, od_all):
    return pl.pallas_call(
        _scale_body,
        grid=(N_ALL // _BLK,),
        in_specs=[
            pl.BlockSpec((_BLK, EMBED_DIM), lambda i: (i, 0)),
            pl.BlockSpec((_BLK, 1), lambda i: (i, 0)),
        ],
        out_specs=pl.BlockSpec((_BLK, EMBED_DIM), lambda i: (i, 0)),
        out_shape=jax.ShapeDtypeStruct((N_ALL, EMBED_DIM), _F32),
    )(x_all, od_all.reshape(N_ALL, 1))


def _agg_body(h_all, eim, eis, eit,
              agg_all,
              spm, sidx, didxl, didxs, rows, db,
              si0, si1, sg0, sg1, ss0, ss1):
    cid = lax.axis_index("c")
    sid = lax.axis_index("s")
    sem_i = [si0, si1]
    sem_g = [sg0, sg1]
    sem_s = [ss0, ss1]

    def zero_spm():
        for j in range(125):
            for k in range(2):
                db[j, pl.ds(k * 16, 16)] = jnp.zeros((16,), _F32)
        for k in range(25):
            c = sid + 16 * k
            pltpu.sync_copy(db, spm.at[pl.ds(pl.multiple_of(c * 125, 125), 125)])

    zero_spm()
    plsc.subcore_barrier()

    def phase(edge_ref, nsuper, masked, src_off):
        base_vec = jnp.full((16,), cid * N_SIDE, _I32)
        dummy = jnp.full((16,), N_SIDE, _I32)
        soffv = jnp.full((16,), src_off, _I32)

        def issue_idx(c, s):
            @pl.when(c < nsuper)
            def _():
                b = pl.multiple_of(c * SUP3, SUP3)
                for j in range(J3):
                    pltpu.async_copy(
                        edge_ref.at[0, pl.ds(b + j * SUB3, SUB3)], sidx.at[s, j],
                        sem_i[s],
                    )
                    pltpu.async_copy(
                        edge_ref.at[1, pl.ds(b + j * SUB3, SUB3)], didxl.at[s, j],
                        sem_i[s],
                    )

        def wait_idx(c, s):
            @pl.when(c < nsuper)
            def _():
                b = pl.multiple_of(c * SUP3, SUP3)
                for j in range(J3):
                    pltpu.make_async_copy(
                        edge_ref.at[0, pl.ds(b + j * SUB3, SUB3)], sidx.at[s, j],
                        sem_i[s],
                    ).wait()
                    pltpu.make_async_copy(
                        edge_ref.at[1, pl.ds(b + j * SUB3, SUB3)], didxl.at[s, j],
                        sem_i[s],
                    ).wait()

        def process(c, s):
            @pl.when(c < nsuper)
            def _():
                if src_off:
                    for j in range(J3):
                        for q in range(SUB3 // 16):
                            sidx[s, j, pl.ds(q * 16, 16)] = (
                                sidx[s, j, pl.ds(q * 16, 16)] + soffv
                            )
                for j in range(J3):
                    pltpu.async_copy(h_all.at[sidx.at[s, j]], rows.at[s, j], sem_g[s])
                for j in range(J3):
                    for q in range(SUB3 // 16):
                        d = didxl[s, j, pl.ds(q * 16, 16)]
                        if masked:
                            inr = (d >= base_vec) & (d < base_vec + N_SIDE)
                            d = jnp.where(inr, d - base_vec, dummy)
                        didxs[s, j, pl.ds(q * 16, 16)] = d
                for j in range(J3):
                    pltpu.make_async_copy(
                        h_all.at[sidx.at[s, j]], rows.at[s, j], sem_g[s]
                    ).wait()
                for j in range(J3):
                    pltpu.async_copy(
                        rows.at[s, j], spm.at[didxs.at[s, j]], sem_s[s], add=True
                    )

        def wait_sc(c, s):
            @pl.when(c < nsuper)
            def _():
                for j in range(J3):
                    pltpu.make_async_copy(
                        rows.at[s, j], spm.at[didxs.at[s, j]], sem_s[s]
                    ).wait()

        trips = (nsuper + 15) // 16
        trips2 = (trips + 1) // 2

        issue_idx(sid, 0)

        def lbody(k2, carry):
            c0 = sid + 16 * (2 * k2)
            c1 = c0 + 16
            c2 = c0 + 32
            wait_idx(c0, 0)
            issue_idx(c1, 1)

            @pl.when(k2 > 0)
            def _():
                wait_sc(c0 - 32, 0)

            process(c0, 0)
            wait_idx(c1, 1)
            issue_idx(c2, 0)

            @pl.when(k2 > 0)
            def _():
                wait_sc(c1 - 32, 1)

            process(c1, 1)
            return carry

        lax.fori_loop(0, trips2, lbody, 0)
        wait_sc(sid + 16 * (2 * trips2 - 2), 0)
        wait_sc(sid + 16 * (2 * trips2 - 1), 1)

    phase(eim, E_MAIN // SUP3, True, 0)
    plsc.subcore_barrier()

    for k in range(25):
        r0 = sid * 3125 + k * 125
        pltpu.sync_copy(spm.at[pl.ds(r0, 125)], db)
        pltpu.sync_copy(db, agg_all.at[pl.ds(cid * N_SIDE + r0, 125)])

    plsc.subcore_barrier()
    zero_spm()
    plsc.subcore_barrier()

    @pl.when(cid == 0)
    def _():
        phase(eis, E_SIDE // SUP3, False, N_MAIN)

    @pl.when(cid == 1)
    def _():
        phase(eit, E_SIDE // SUP3, False, N_MAIN + N_SIDE)

    plsc.subcore_barrier()

    def drain_side(agg_off):
        for k in range(25):
            r0 = sid * 3125 + k * 125
            pltpu.sync_copy(spm.at[pl.ds(r0, 125)], db)
            pltpu.sync_copy(db, agg_all.at[pl.ds(agg_off + r0, 125)])

    @pl.when(cid == 0)
    def _():
        drain_side(N_MAIN)

    @pl.when(cid == 1)
    def _():
        drain_side(N_MAIN + N_SIDE)


_agg_kernel = pl.kernel(
    _agg_body,
    out_type=jax.ShapeDtypeStruct((N_ALL, EMBED_DIM), _F32),
    mesh=_MESH,
    compiler_params=pltpu.CompilerParams(use_tc_tiling_on_sc=False),
    scratch_types=[
        pltpu.VMEM_SHARED((N_SIDE + 1, EMBED_DIM), _F32),
        pltpu.VMEM((2, J3, SUB3), _I32),
        pltpu.VMEM((2, J3, SUB3), _I32),
        pltpu.VMEM((2, J3, SUB3), _I32),
        pltpu.VMEM((2, J3, SUB3, EMBED_DIM), _F32),
        pltpu.VMEM((125, EMBED_DIM), _F32),
        pltpu.SemaphoreType.DMA,
        pltpu.SemaphoreType.DMA,
        pltpu.SemaphoreType.DMA,
        pltpu.SemaphoreType.DMA,
        pltpu.SemaphoreType.DMA,
        pltpu.SemaphoreType.DMA,
    ],
)


_BLK = 5000
_G_MAIN = N_MAIN // _BLK
_G_SRC = _G_MAIN + N_SIDE // _BLK
_G_ALL = N_ALL // _BLK


def _mm_body(agg_ref, d_ref, w_ref, b_ref, fcw_ref, fcb_ref, dw_ref, db_ref,
             o_b, o_d, o_s, o_t, acc_ref):
    i = pl.program_id(0)
    d = d_ref[...]
    norm = jnp.where(d > 0, lax.rsqrt(jnp.maximum(d, 1.0)), 0.0)
    z = jnp.dot(agg_ref[...] * norm, w_ref[...], preferred_element_type=_F32)
    z = jnp.maximum(z + b_ref[...], 0.0)
    m = jnp.max(z, axis=0, keepdims=True)

    for gid, (lo, hi) in enumerate(((0, _G_MAIN), (_G_MAIN, _G_SRC),
                                    (_G_SRC, _G_ALL))):
        @pl.when(i == lo)
        def _():
            acc_ref[pl.ds(gid, 1), :] = m

        @pl.when((i > lo) & (i < hi))
        def _():
            acc_ref[pl.ds(gid, 1), :] = jnp.maximum(acc_ref[pl.ds(gid, 1), :], m)

    @pl.when(i == _G_ALL - 1)
    def _():
        fcw = fcw_ref[...]
        fcb = fcb_ref[...]
        xb = jnp.dot(acc_ref[pl.ds(0, 1), :], fcw, preferred_element_type=_F32) + fcb
        xs = jnp.dot(acc_ref[pl.ds(1, 1), :], fcw, preferred_element_type=_F32) + fcb
        xt = jnp.dot(acc_ref[pl.ds(2, 1), :], fcw, preferred_element_type=_F32) + fcb
        logit = jnp.dot(xb, dw_ref[...], preferred_element_type=_F32) + db_ref[...]
        o_b[...] = xb
        o_d[...] = 1.0 / (1.0 + jnp.exp(-logit))
        o_s[...] = xs
        o_t[...] = xt


def _max_head(agg_all, id_all, w_p, b_p, fcw_p, fcb2, dw, db2):
    return pl.pallas_call(
        _mm_body,
        grid=(_G_ALL,),
        in_specs=[
            pl.BlockSpec((_BLK, EMBED_DIM), lambda i: (i, 0)),
            pl.BlockSpec((_BLK, 1), lambda i: (i, 0)),
            pl.BlockSpec((EMBED_DIM, HID_PAD), lambda i: (0, 0)),
            pl.BlockSpec((1, HID_PAD), lambda i: (0, 0)),
            pl.BlockSpec((HID_PAD, 32), lambda i: (0, 0)),
            pl.BlockSpec((1, 32), lambda i: (0, 0)),
            pl.BlockSpec((32, 1), lambda i: (0, 0)),
            pl.BlockSpec((1, 1), lambda i: (0, 0)),
        ],
        out_specs=(
            pl.BlockSpec((1, 32), lambda i: (0, 0)),
            pl.BlockSpec((1, 1), lambda i: (0, 0)),
            pl.BlockSpec((1, 32), lambda i: (0, 0)),
            pl.BlockSpec((1, 32), lambda i: (0, 0)),
        ),
        out_shape=(
            jax.ShapeDtypeStruct((1, 32), _F32),
            jax.ShapeDtypeStruct((1, 1), _F32),
            jax.ShapeDtypeStruct((1, 32), _F32),
            jax.ShapeDtypeStruct((1, 32), _F32),
        ),
        scratch_shapes=[pltpu.VMEM((8, HID_PAD), _F32)],
    )(agg_all, id_all.reshape(N_ALL, 1), w_p, b_p, fcw_p, fcb2, dw, db2)


def kernel(feat, edge_index, feat_src, edge_index_src, feat_tar, edge_index_tar,
           embed, gcn_w, gcn_b, fc_w, fc_b, def_w, def_b):
    od_all, id_all = _deg_kernel(edge_index, edge_index_src, edge_index_tar)
    x_all = _h_kernel(feat, feat_src, feat_tar, embed)
    h_all = _scale(x_all, od_all)
    agg_all = _agg_kernel(h_all, edge_index, edge_index_src, edge_index_tar)

    w_p = jnp.pad(gcn_w, ((0, 0), (0, HID_PAD - HIDDEN)))
    b_p = jnp.pad(gcn_b, (0, HID_PAD - HIDDEN)).reshape(1, HID_PAD)
    fcw_p = jnp.pad(fc_w, ((0, HID_PAD - HIDDEN), (0, 0)))
    fcb2 = fc_b.reshape(1, 32)
    db2 = def_b.reshape(1, 1)

    x_batch_mmd, defect_prob, x_src_mmd, x_tar_mmd = _max_head(
        agg_all, id_all, w_p, b_p, fcw_p, fcb2, def_w, db2
    )
    return (x_batch_mmd, defect_prob, x_src_mmd, x_tar_mmd)


# speedup vs baseline: 14.2789x; 1.2968x over previous
import jax
import jax.numpy as jnp
from jax import lax
from jax.experimental import pallas as pl
from jax.experimental.pallas import tpu as pltpu
from jax.experimental.pallas import tpu_sc as plsc

N_MAIN = 100000
N_SIDE = 50000
N_ALL = N_MAIN + 2 * N_SIDE
E_MAIN = 1600000
E_SIDE = 800000
EMBED_DIM = 32
HIDDEN = 100
HID_PAD = 128

J = 5
SUP = J * 128
J3 = 4
SUB3 = 80
SUP3 = J3 * SUB3
GB = 80

_MESH = plsc.VectorSubcoreMesh(
    core_axis_name="c", subcore_axis_name="s", num_cores=2, num_subcores=16
)

_F32 = jnp.float32
_I32 = jnp.int32


def _fill(ref, n, val, dtype):
    vec = jnp.full((16,), val, dtype)
    for j in range(n // 16):
        ref[pl.ds(j * 16, 16)] = vec


def _deg_body(eim, eis, eit,
              od_all, id_all,
              spm, il, isx, ones_v, zb, db, si0, si1, ss0, ss1):
    cid = lax.axis_index("c")
    sid = lax.axis_index("s")
    sem_i = [si0, si1]
    sem_s = [ss0, ss1]

    _fill(ones_v, 128, 1.0, _F32)
    _fill(zb, 2000, 0.0, _F32)

    for k in range(7):
        c = sid + 16 * k

        @pl.when(c < 100)
        def _():
            pltpu.sync_copy(zb, spm.at[pl.ds(pl.multiple_of(c * 2000, 2000), 2000)])

    plsc.subcore_barrier()

    def scatter_deg(edge_ref, row, off, nsuper):
        offv = jnp.full((16,), off, _I32)

        def issue_idx(c, s):
            @pl.when(c < nsuper)
            def _():
                b = pl.multiple_of(c * SUP, 128)
                for j in range(J):
                    pltpu.async_copy(
                        edge_ref.at[row, pl.ds(b + j * 128, 128)], il.at[s, j],
                        sem_i[s],
                    )

        def wait_idx(c, s):
            @pl.when(c < nsuper)
            def _():
                b = pl.multiple_of(c * SUP, 128)
                for j in range(J):
                    pltpu.make_async_copy(
                        edge_ref.at[row, pl.ds(b + j * 128, 128)], il.at[s, j],
                        sem_i[s],
                    ).wait()

        def process(c, s):
            @pl.when(c < nsuper)
            def _():
                for j in range(J):
                    for q in range(8):
                        isx[s, j, pl.ds(q * 16, 16)] = (
                            il[s, j, pl.ds(q * 16, 16)] + offv
                        )
                for j in range(J):
                    pltpu.async_copy(
                        ones_v, spm.at[isx.at[s, j]], sem_s[s], add=True
                    )

        def wait_sc(c, s):
            @pl.when(c < nsuper)
            def _():
                for j in range(J):
                    pltpu.make_async_copy(
                        ones_v, spm.at[isx.at[s, j]], sem_s[s]
                    ).wait()

        trips = (nsuper + 15) // 16
        trips2 = (trips + 1) // 2

        issue_idx(sid, 0)

        def lbody(k2, carry):
            c0 = sid + 16 * (2 * k2)
            c1 = c0 + 16
            c2 = c0 + 32
            wait_idx(c0, 0)
            issue_idx(c1, 1)

            @pl.when(k2 > 0)
            def _():
                wait_sc(c0 - 32, 0)

            process(c0, 0)
            wait_idx(c1, 1)
            issue_idx(c2, 0)

            @pl.when(k2 > 0)
            def _():
                wait_sc(c1 - 32, 1)

            process(c1, 1)
            return carry

        lax.fori_loop(0, trips2, lbody, 0)
        wait_sc(sid + 16 * (2 * trips2 - 2), 0)
        wait_sc(sid + 16 * (2 * trips2 - 1), 1)

    @pl.when(cid == 0)
    def _():
        scatter_deg(eim, 0, 0, E_MAIN // SUP)
        scatter_deg(eim, 1, 100000, E_MAIN // SUP)

    @pl.when(cid == 1)
    def _():
        scatter_deg(eis, 0, 0, E_SIDE // SUP)
        scatter_deg(eis, 1, 50000, E_SIDE // SUP)
        scatter_deg(eit, 0, 100000, E_SIDE // SUP)
        scatter_deg(eit, 1, 150000, E_SIDE // SUP)

    plsc.subcore_barrier()

    def drain(spm_off, hbm_ref, hbm_off, n):
        nch = n // 2000
        for k in range((nch + 15) // 16):
            c = sid + 16 * k

            @pl.when(c < nch)
            def _():
                b = pl.multiple_of(c * 2000, 2000)
                pltpu.sync_copy(spm.at[pl.ds(spm_off + b, 2000)], db)
                pltpu.sync_copy(db, hbm_ref.at[pl.ds(hbm_off + b, 2000)])

    @pl.when(cid == 0)
    def _():
        drain(0, od_all, 0, N_MAIN)
        drain(100000, id_all, 0, N_MAIN)

    @pl.when(cid == 1)
    def _():
        drain(0, od_all, 100000, N_SIDE)
        drain(50000, id_all, 100000, N_SIDE)
        drain(100000, od_all, 150000, N_SIDE)
        drain(150000, id_all, 150000, N_SIDE)


_deg_kernel = pl.kernel(
    _deg_body,
    out_type=(
        jax.ShapeDtypeStruct((N_ALL,), _F32),
        jax.ShapeDtypeStruct((N_ALL,), _F32),
    ),
    mesh=_MESH,
    compiler_params=pltpu.CompilerParams(use_tc_tiling_on_sc=False),
    scratch_types=[
        pltpu.VMEM_SHARED((200000,), _F32),
        pltpu.VMEM((2, J, 128), _I32),
        pltpu.VMEM((2, J, 128), _I32),
        pltpu.VMEM((128,), _F32),
        pltpu.VMEM((2000,), _F32),
        pltpu.VMEM((2000,), _F32),
        pltpu.SemaphoreType.DMA,
        pltpu.SemaphoreType.DMA,
        pltpu.SemaphoreType.DMA,
        pltpu.SemaphoreType.DMA,
    ],
)


def _h_body(featm, feats, featt, embed, x_all, idx_v, rows_v, sem):
    cid = lax.axis_index("c")
    sid = lax.axis_index("s")
    gw = sid * 2 + cid

    def phase(feat_ref, n, x_off):
        nchunks = n // GB
        trips = (nchunks + 31) // 32

        def body(i, carry):
            c = gw + 32 * i

            @pl.when(c < nchunks)
            def _():
                base = pl.multiple_of(c * GB, GB)
                pltpu.sync_copy(feat_ref.at[pl.ds(base, GB)], idx_v)
                pltpu.async_copy(embed.at[idx_v], rows_v, sem).wait()
                pltpu.sync_copy(rows_v, x_all.at[pl.ds(x_off + base, GB)])

            return carry

        lax.fori_loop(0, trips, body, 0)

    phase(featm, N_MAIN, 0)
    phase(feats, N_SIDE, 100000)
    phase(featt, N_SIDE, 150000)


_h_kernel = pl.kernel(
    _h_body,
    out_type=jax.ShapeDtypeStruct((N_ALL, EMBED_DIM), _F32),
    mesh=_MESH,
    compiler_params=pltpu.CompilerParams(use_tc_tiling_on_sc=False),
    scratch_types=[
        pltpu.VMEM((GB,), _I32),
        pltpu.VMEM((GB, EMBED_DIM), _F32),
        pltpu.SemaphoreType.DMA,
    ],
)


def _scale_body(x_ref, d_ref, o_ref):
    d = d_ref[...]
    norm = jnp.where(d > 0, lax.rsqrt(jnp.maximum(d, 1.0)), 0.0)
    o_ref[...] = x_ref[...] * norm


def _scale(x_all, od_all):
    return pl.pallas_call(
        _scale_body,
        grid=(N_ALL // _BLK,),
        in_specs=[
            pl.BlockSpec((_BLK, EMBED_DIM), lambda i: (i, 0)),
            pl.BlockSpec((_BLK, 1), lambda i: (i, 0)),
        ],
        out_specs=pl.BlockSpec((_BLK, EMBED_DIM), lambda i: (i, 0)),
        out_shape=jax.ShapeDtypeStruct((N_ALL, EMBED_DIM), _F32),
    )(x_all, od_all.reshape(N_ALL, 1))


def _agg_body(h2, eim, eis, eit,
              agg2,
              spm, sidx, sidxc, didxl, didxs, rows, db,
              si0, si1, sg0, sg1, ss0, ss1):
    cid = lax.axis_index("c")
    sid = lax.axis_index("s")
    sem_i = [si0, si1]
    sem_g = [sg0, sg1]
    sem_s = [ss0, ss1]
    NV = SUB3 // 16

    def zero_spm():
        for j in range(125):
            db[j, pl.ds(0, 16)] = jnp.zeros((16,), _F32)
        for k in range(50):
            c = sid + 16 * k
            pltpu.sync_copy(db, spm.at[pl.ds(pl.multiple_of(c * 125, 125), 125)])

    zero_spm()
    plsc.subcore_barrier()

    def phase(edge_ref, nsuper, hvecs, dvecs):
        nq = len(hvecs)

        def issue_idx(c, s):
            @pl.when(c < nsuper)
            def _():
                b = pl.multiple_of(c * SUP3, SUP3)
                for j in range(J3):
                    pltpu.async_copy(
                        edge_ref.at[0, pl.ds(b + j * SUB3, SUB3)], sidx.at[s, j],
                        sem_i[s],
                    )
                    pltpu.async_copy(
                        edge_ref.at[1, pl.ds(b + j * SUB3, SUB3)], didxl.at[s, j],
                        sem_i[s],
                    )

        def wait_idx(c, s):
            @pl.when(c < nsuper)
            def _():
                b = pl.multiple_of(c * SUP3, SUP3)
                for j in range(J3):
                    pltpu.make_async_copy(
                        edge_ref.at[0, pl.ds(b + j * SUB3, SUB3)], sidx.at[s, j],
                        sem_i[s],
                    ).wait()
                    pltpu.make_async_copy(
                        edge_ref.at[1, pl.ds(b + j * SUB3, SUB3)], didxl.at[s, j],
                        sem_i[s],
                    ).wait()

        def process(c, s):
            @pl.when(c < nsuper)
            def _():
                for j in range(J3):
                    for q in range(nq):
                        for v in range(NV):
                            t = sidx[s, j, pl.ds(v * 16, 16)]
                            sidxc[s, j, q, pl.ds(v * 16, 16)] = t + t + hvecs[q]
                for j in range(J3):
                    for q in range(nq):
                        pltpu.async_copy(
                            h2.at[sidxc.at[s, j, q]], rows.at[s, j, q], sem_g[s]
                        )
                for j in range(J3):
                    for q in range(nq):
                        for v in range(NV):
                            didxs[s, j, q, pl.ds(v * 16, 16)] = (
                                didxl[s, j, pl.ds(v * 16, 16)] + dvecs[q]
                            )
                for j in range(J3):
                    for q in range(nq):
                        pltpu.make_async_copy(
                            h2.at[sidxc.at[s, j, q]], rows.at[s, j, q], sem_g[s]
                        ).wait()
                for j in range(J3):
                    for q in range(nq):
                        pltpu.async_copy(
                            rows.at[s, j, q], spm.at[didxs.at[s, j, q]],
                            sem_s[s], add=True,
                        )

        def wait_sc(c, s):
            @pl.when(c < nsuper)
            def _():
                for j in range(J3):
                    for q in range(nq):
                        pltpu.make_async_copy(
                            rows.at[s, j, q], spm.at[didxs.at[s, j, q]], sem_s[s]
                        ).wait()

        trips = (nsuper + 15) // 16
        trips2 = (trips + 1) // 2

        issue_idx(sid, 0)

        def lbody(k2, carry):
            c0 = sid + 16 * (2 * k2)
            c1 = c0 + 16
            c2 = c0 + 32
            wait_idx(c0, 0)
            issue_idx(c1, 1)

            @pl.when(k2 > 0)
            def _():
                wait_sc(c0 - 32, 0)

            process(c0, 0)
            wait_idx(c1, 1)
            issue_idx(c2, 0)

            @pl.when(k2 > 0)
            def _():
                wait_sc(c1 - 32, 1)

            process(c1, 1)
            return carry

        lax.fori_loop(0, trips2, lbody, 0)
        wait_sc(sid + 16 * (2 * trips2 - 2), 0)
        wait_sc(sid + 16 * (2 * trips2 - 1), 1)

    zero16 = jnp.zeros((16,), _I32)
    cid_vec = jnp.full((16,), 0, _I32) + cid.astype(_I32)

    phase(eim, E_MAIN // SUP3, [cid_vec], [zero16])
    plsc.subcore_barrier()

    for k in range(50):
        r0 = sid * 6250 + k * 125
        pltpu.sync_copy(spm.at[pl.ds(r0, 125)], db)
        pltpu.sync_copy(db, agg2.at[cid, pl.ds(r0, 125)])

    plsc.subcore_barrier()
    zero_spm()
    plsc.subcore_barrier()

    half_off = [jnp.full((16,), 0, _I32), jnp.full((16,), N_SIDE, _I32)]

    @pl.when(cid == 0)
    def _():
        phase(eis, E_SIDE // SUP3,
              [jnp.full((16,), 2 * N_MAIN + q, _I32) for q in range(2)], half_off)

    @pl.when(cid == 1)
    def _():
        phase(eit, E_SIDE // SUP3,
              [jnp.full((16,), 2 * (N_MAIN + N_SIDE) + q, _I32) for q in range(2)],
              half_off)

    plsc.subcore_barrier()

    def drain_side(agg_off):
        for q in range(2):
            for k in range(25):
                r0 = sid * 3125 + k * 125
                pltpu.sync_copy(spm.at[pl.ds(q * N_SIDE + r0, 125)], db)
                pltpu.sync_copy(db, agg2.at[q, pl.ds(agg_off + r0, 125)])

    @pl.when(cid == 0)
    def _():
        drain_side(N_MAIN)

    @pl.when(cid == 1)
    def _():
        drain_side(N_MAIN + N_SIDE)


_agg_kernel = pl.kernel(
    _agg_body,
    out_type=jax.ShapeDtypeStruct((2, N_ALL, 16), _F32),
    mesh=_MESH,
    compiler_params=pltpu.CompilerParams(use_tc_tiling_on_sc=False),
    scratch_types=[
        pltpu.VMEM_SHARED((2 * N_SIDE, 16), _F32),
        pltpu.VMEM((2, J3, SUB3), _I32),
        pltpu.VMEM((2, J3, 2, SUB3), _I32),
        pltpu.VMEM((2, J3, SUB3), _I32),
        pltpu.VMEM((2, J3, 2, SUB3), _I32),
        pltpu.VMEM((2, J3, 2, SUB3, 16), _F32),
        pltpu.VMEM((125, 16), _F32),
        pltpu.SemaphoreType.DMA,
        pltpu.SemaphoreType.DMA,
        pltpu.SemaphoreType.DMA,
        pltpu.SemaphoreType.DMA,
        pltpu.SemaphoreType.DMA,
        pltpu.SemaphoreType.DMA,
    ],
)


_BLK = 5000
_G_MAIN = N_MAIN // _BLK
_G_SRC = _G_MAIN + N_SIDE // _BLK
_G_ALL = N_ALL // _BLK


def _mm_body(agg_ref, d_ref, w_ref, b_ref, fcw_ref, fcb_ref, dw_ref, db_ref,
             o_b, o_d, o_s, o_t, acc_ref):
    i = pl.program_id(0)
    d = d_ref[...]
    norm = jnp.where(d > 0, lax.rsqrt(jnp.maximum(d, 1.0)), 0.0)
    w = w_ref[...]
    a = agg_ref[...]
    z = jnp.dot(a[0] * norm, w[0:16, :], preferred_element_type=_F32)
    z = z + jnp.dot(a[1] * norm, w[16:32, :], preferred_element_type=_F32)
    z = jnp.maximum(z + b_ref[...], 0.0)
    m = jnp.max(z, axis=0, keepdims=True)

    for gid, (lo, hi) in enumerate(((0, _G_MAIN), (_G_MAIN, _G_SRC),
                                    (_G_SRC, _G_ALL))):
        @pl.when(i == lo)
        def _():
            acc_ref[pl.ds(gid, 1), :] = m

        @pl.when((i > lo) & (i < hi))
        def _():
            acc_ref[pl.ds(gid, 1), :] = jnp.maximum(acc_ref[pl.ds(gid, 1), :], m)

    @pl.when(i == _G_ALL - 1)
    def _():
        fcw = fcw_ref[...]
        fcb = fcb_ref[...]
        xb = jnp.dot(acc_ref[pl.ds(0, 1), :], fcw, preferred_element_type=_F32) + fcb
        xs = jnp.dot(acc_ref[pl.ds(1, 1), :], fcw, preferred_element_type=_F32) + fcb
        xt = jnp.dot(acc_ref[pl.ds(2, 1), :], fcw, preferred_element_type=_F32) + fcb
        logit = jnp.dot(xb, dw_ref[...], preferred_element_type=_F32) + db_ref[...]
        o_b[...] = xb
        o_d[...] = 1.0 / (1.0 + jnp.exp(-logit))
        o_s[...] = xs
        o_t[...] = xt


def _max_head(agg_all, id_all, w_p, b_p, fcw_p, fcb2, dw, db2):
    return pl.pallas_call(
        _mm_body,
        grid=(_G_ALL,),
        in_specs=[
            pl.BlockSpec((2, _BLK, 16), lambda i: (0, i, 0)),
            pl.BlockSpec((_BLK, 1), lambda i: (i, 0)),
            pl.BlockSpec((EMBED_DIM, HID_PAD), lambda i: (0, 0)),
            pl.BlockSpec((1, HID_PAD), lambda i: (0, 0)),
            pl.BlockSpec((HID_PAD, 32), lambda i: (0, 0)),
            pl.BlockSpec((1, 32), lambda i: (0, 0)),
            pl.BlockSpec((32, 1), lambda i: (0, 0)),
            pl.BlockSpec((1, 1), lambda i: (0, 0)),
        ],
        out_specs=(
            pl.BlockSpec((1, 32), lambda i: (0, 0)),
            pl.BlockSpec((1, 1), lambda i: (0, 0)),
            pl.BlockSpec((1, 32), lambda i: (0, 0)),
            pl.BlockSpec((1, 32), lambda i: (0, 0)),
        ),
        out_shape=(
            jax.ShapeDtypeStruct((1, 32), _F32),
            jax.ShapeDtypeStruct((1, 1), _F32),
            jax.ShapeDtypeStruct((1, 32), _F32),
            jax.ShapeDtypeStruct((1, 32), _F32),
        ),
        scratch_shapes=[pltpu.VMEM((8, HID_PAD), _F32)],
    )(agg_all, id_all.reshape(N_ALL, 1), w_p, b_p, fcw_p, fcb2, dw, db2)


def kernel(feat, edge_index, feat_src, edge_index_src, feat_tar, edge_index_tar,
           embed, gcn_w, gcn_b, fc_w, fc_b, def_w, def_b):
    od_all, id_all = _deg_kernel(edge_index, edge_index_src, edge_index_tar)
    x_all = _h_kernel(feat, feat_src, feat_tar, embed)
    h_all = _scale(x_all, od_all)
    h2 = h_all.reshape(2 * N_ALL, 16)
    agg_all = _agg_kernel(h2, edge_index, edge_index_src, edge_index_tar)

    w_p = jnp.pad(gcn_w, ((0, 0), (0, HID_PAD - HIDDEN)))
    b_p = jnp.pad(gcn_b, (0, HID_PAD - HIDDEN)).reshape(1, HID_PAD)
    fcw_p = jnp.pad(fc_w, ((0, HID_PAD - HIDDEN), (0, 0)))
    fcb2 = fc_b.reshape(1, 32)
    db2 = def_b.reshape(1, 1)

    x_batch_mmd, defect_prob, x_src_mmd, x_tar_mmd = _max_head(
        agg_all, id_all, w_p, b_p, fcw_p, fcb2, def_w, db2
    )
    return (x_batch_mmd, defect_prob, x_src_mmd, x_tar_mmd)
